# baseline (speedup 1.0000x reference)
import functools

import jax
import jax.numpy as jnp
from jax import lax
from jax.experimental import pallas as pl
from jax.experimental.pallas import tpu as pltpu
from jax.experimental.pallas import tpu_sc as plsc

N = 10000
E = 320000
D_IN = 128
DH = 64

NC = 2
NS = 16
NW = NC * NS
CH = 80
NCH = 128
ZCH = 32
EW = CH * NCH
EP = NW * EW
N_PAD = 10240
ZB = N_PAD // NS

_MESH = plsc.VectorSubcoreMesh(core_axis_name="c", subcore_axis_name="s")

BM = 2000
GRID = N // BM
BME = 5000
GRIDE = N // BME


def _fill_rows(ref, nrows, value):
    @pl.loop(0, nrows)
    def _(i):
        ref.at[pl.ds(i, 1), :][...] = jnp.full((1, 16), value, jnp.float32)


@functools.partial(
    pl.kernel,
    out_type=jax.ShapeDtypeStruct((NC, N_PAD, 16), jnp.float32),
    mesh=_MESH,
    scratch_types=[
        pltpu.VMEM((NCH, CH), jnp.int32),
        pltpu.VMEM((CH, 16), jnp.float32),
        pltpu.VMEM((ZB, 16), jnp.float32),
        pltpu.VMEM_SHARED((N_PAD, 16), jnp.float32),
        pltpu.SemaphoreType.DMA,
    ],
    compiler_params=pltpu.CompilerParams(use_tc_tiling_on_sc=False),
)
def _deg_kernel(ei4_hbm, out_hbm, didx, ones_v, zeros_v, acc, ssem):
    c = lax.axis_index("c")
    s = lax.axis_index("s")
    wid = s * NC + c

    _fill_rows(ones_v, CH, 1.0)
    _fill_rows(zeros_v, ZB, 0.0)
    pltpu.sync_copy(zeros_v, acc.at[pl.ds(s * ZB, ZB)])
    plsc.subcore_barrier()

    pltpu.sync_copy(ei4_hbm.at[1, wid], didx)

    @pl.loop(0, NCH, step=8)
    def _(g):
        for j in range(8):
            pltpu.async_copy(ones_v, acc.at[didx.at[g + j]], ssem, add=True)
        for j in range(8):
            pltpu.make_async_copy(ones_v, acc.at[didx.at[g + j]], ssem).wait()

    plsc.subcore_barrier()
    pltpu.sync_copy(acc.at[pl.ds(s * ZB, ZB)],
                    out_hbm.at[c, pl.ds(s * ZB, ZB)])


@functools.partial(
    pl.kernel,
    out_type=jax.ShapeDtypeStruct((NC, N_PAD, DH), jnp.float32),
    mesh=_MESH,
    scratch_types=[
        pltpu.VMEM((NCH, CH), jnp.int32),
        pltpu.VMEM((NCH, CH), jnp.int32),
        pltpu.VMEM((4, CH, DH), jnp.float32),
        pltpu.VMEM((ZCH, DH), jnp.float32),
        pltpu.VMEM_SHARED((N_PAD, DH), jnp.float32),
        pltpu.VMEM_SHARED((N_PAD, DH), jnp.float32),
        pltpu.SemaphoreType.DMA((4,)),
        pltpu.SemaphoreType.DMA((4,)),
    ],
    compiler_params=pltpu.CompilerParams(use_tc_tiling_on_sc=False),
)
def _agg_kernel(hs_hbm, ei4_hbm, out_hbm,
                sidx, didx, rows, zeros_v, acc, hs_sh, gsem, ssem):
    c = lax.axis_index("c")
    s = lax.axis_index("s")
    wid = s * NC + c

    @pl.loop(0, ZCH)
    def _(i):
        @pl.loop(0, DH, step=16)
        def _(j):
            zeros_v.at[pl.ds(i, 1), pl.ds(j, 16)][...] = jnp.zeros(
                (1, 16), jnp.float32)

    pltpu.sync_copy(hs_hbm.at[pl.ds(s * (N // NS), N // NS)],
                    hs_sh.at[pl.ds(s * (N // NS), N // NS)])
    for z in range(ZB // ZCH):
        pltpu.sync_copy(zeros_v, acc.at[pl.ds(s * ZB + z * ZCH, ZCH)])
    plsc.subcore_barrier()

    pltpu.sync_copy(ei4_hbm.at[0, wid], sidx)
    pltpu.sync_copy(ei4_hbm.at[1, wid], didx)

    def fire_gather(k, b):
        pltpu.async_copy(hs_sh.at[sidx.at[k]], rows.at[b], gsem.at[b])

    def wait_gather(k, b):
        pltpu.make_async_copy(hs_sh.at[sidx.at[k]], rows.at[b],
                              gsem.at[b]).wait()

    def fire_scatter(k, b):
        pltpu.async_copy(rows.at[b], acc.at[didx.at[k]], ssem.at[b], add=True)

    def wait_scatter(k, b):
        pltpu.make_async_copy(rows.at[b], acc.at[didx.at[k]],
                              ssem.at[b]).wait()

    fire_gather(0, 0)
    fire_gather(1, 1)
    fire_gather(2, 2)
    wait_gather(0, 0)
    fire_scatter(0, 0)
    fire_gather(3, 3)
    wait_gather(1, 1)
    fire_scatter(1, 1)
    wait_scatter(0, 0)
    fire_gather(4, 0)
    wait_gather(2, 2)
    fire_scatter(2, 2)
    wait_scatter(1, 1)
    fire_gather(5, 1)
    wait_gather(3, 3)
    fire_scatter(3, 3)

    @pl.loop(4, NCH - 4, step=4)
    def _(kk):
        for j in range(4):
            k = kk + j
            bn = (j + 2) % 4
            wait_scatter(k - 2, bn)
            fire_gather(k + 2, bn)
            wait_gather(k, j)
            fire_scatter(k, j)

    k0 = NCH - 4
    wait_scatter(k0 - 2, 2)
    fire_gather(k0 + 2, 2)
    wait_gather(k0, 0)
    fire_scatter(k0, 0)
    wait_scatter(k0 - 1, 3)
    fire_gather(k0 + 3, 3)
    wait_gather(k0 + 1, 1)
    fire_scatter(k0 + 1, 1)
    wait_gather(k0 + 2, 2)
    fire_scatter(k0 + 2, 2)
    wait_gather(k0 + 3, 3)
    fire_scatter(k0 + 3, 3)
    wait_scatter(k0, 0)
    wait_scatter(k0 + 1, 1)
    wait_scatter(k0 + 2, 2)
    wait_scatter(k0 + 3, 3)

    plsc.subcore_barrier()
    pltpu.sync_copy(acc.at[pl.ds(s * ZB, ZB)],
                    out_hbm.at[c, pl.ds(s * ZB, ZB)])


_PREC = lax.Precision.HIGHEST


def _tc0_body(x_ref, w_ref, h_ref):
    h_ref[...] = jnp.dot(x_ref[...], w_ref[...], precision=_PREC,
                         preferred_element_type=jnp.float32)


def _tc1_body(h_ref, dp_ref, hs_ref, dinv_ref):
    deg = dp_ref[0] + dp_ref[1] + 1.0
    dinv = lax.rsqrt(deg)
    hs_ref[...] = h_ref[...] * dinv[:, 0:1]
    dinv_ref[...] = dinv


def _tc2_body(ap_ref, hs_ref, dinv_ref, b_ref, w_ref, out_ref):
    di = dinv_ref[:, 0:1]
    t = di * (ap_ref[0] + ap_ref[1] + hs_ref[...]) + b_ref[...]
    t = jnp.maximum(t, 0.0)
    out_ref[...] = jnp.dot(t, w_ref[...], precision=_PREC,
                           preferred_element_type=jnp.float32) * di


def _tc3_body(ap_ref, hs_ref, dinv_ref, b_ref, w3_ref, b3_ref, out_ref):
    di = dinv_ref[:, 0:1]
    t = di * (ap_ref[0] + ap_ref[1] + hs_ref[...]) + b_ref[...]
    t = jnp.maximum(t, 0.0)
    logits = jnp.dot(t, w3_ref[...], precision=_PREC,
                     preferred_element_type=jnp.float32) + b3_ref[...]
    l0 = logits[:, 0:1]
    l1 = logits[:, 1:2]
    m = jnp.maximum(l0, l1)
    lse = m + jnp.log(jnp.exp(l0 - m) + jnp.exp(l1 - m))
    out_ref[...] = jnp.concatenate([l0 - lse, l1 - lse], axis=1)


_tc0 = pl.pallas_call(
    _tc0_body,
    grid=(GRID,),
    in_specs=[
        pl.BlockSpec((BM, D_IN), lambda i: (i, 0)),
        pl.BlockSpec((D_IN, DH), lambda i: (0, 0)),
    ],
    out_specs=pl.BlockSpec((BM, DH), lambda i: (i, 0)),
    out_shape=jax.ShapeDtypeStruct((N, DH), jnp.float32),
)

_tc1 = pl.pallas_call(
    _tc1_body,
    grid=(GRIDE,),
    in_specs=[
        pl.BlockSpec((BME, DH), lambda i: (i, 0)),
        pl.BlockSpec((NC, BME, 16), lambda i: (0, i, 0)),
    ],
    out_specs=[
        pl.BlockSpec((BME, DH), lambda i: (i, 0)),
        pl.BlockSpec((BME, 16), lambda i: (i, 0)),
    ],
    out_shape=[
        jax.ShapeDtypeStruct((N, DH), jnp.float32),
        jax.ShapeDtypeStruct((N, 16), jnp.float32),
    ],
)

_tc2 = pl.pallas_call(
    _tc2_body,
    grid=(GRIDE,),
    in_specs=[
        pl.BlockSpec((NC, BME, DH), lambda i: (0, i, 0)),
        pl.BlockSpec((BME, DH), lambda i: (i, 0)),
        pl.BlockSpec((BME, 16), lambda i: (i, 0)),
        pl.BlockSpec((1, DH), lambda i: (0, 0)),
        pl.BlockSpec((DH, DH), lambda i: (0, 0)),
    ],
    out_specs=pl.BlockSpec((BME, DH), lambda i: (i, 0)),
    out_shape=jax.ShapeDtypeStruct((N, DH), jnp.float32),
)

_tc3 = pl.pallas_call(
    _tc3_body,
    grid=(GRIDE,),
    in_specs=[
        pl.BlockSpec((NC, BME, DH), lambda i: (0, i, 0)),
        pl.BlockSpec((BME, DH), lambda i: (i, 0)),
        pl.BlockSpec((BME, 16), lambda i: (i, 0)),
        pl.BlockSpec((1, DH), lambda i: (0, 0)),
        pl.BlockSpec((DH, 128), lambda i: (0, 0)),
        pl.BlockSpec((1, 128), lambda i: (0, 0)),
    ],
    out_specs=pl.BlockSpec((BME, 2), lambda i: (i, 0)),
    out_shape=jax.ShapeDtypeStruct((N, 2), jnp.float32),
)


def kernel(x, edge_index, W1, b1, W2, b2, W3, b3):
    ei4 = jnp.pad(edge_index, ((0, 0), (0, EP - E)),
                  constant_values=N).reshape(2, NW, NCH, CH)

    dp = _deg_kernel(ei4)
    h1 = _tc0(x, W1)
    hs1, dinv = _tc1(h1, dp)
    ap1 = _agg_kernel(hs1, ei4)
    hs2 = _tc2(ap1, hs1, dinv, b1.reshape(1, DH), W2)
    ap2 = _agg_kernel(hs2, ei4)
    w3p = jnp.pad(W3, ((0, 0), (0, 128 - W3.shape[1])))
    b3p = jnp.pad(b3, (0, 128 - b3.shape[0])).reshape(1, 128)
    return _tc3(ap2, hs2, dinv, b2.reshape(1, DH), w3p, b3p)


# speedup vs baseline: 1.0385x; 1.0385x over previous
import functools

import jax
import jax.numpy as jnp
from jax import lax
from jax.experimental import pallas as pl
from jax.experimental.pallas import tpu as pltpu
from jax.experimental.pallas import tpu_sc as plsc

N = 10000
E = 320000
D_IN = 128
DH = 64

NC = 2
NS = 16
NW = NC * NS
CH = 80
NCH = 128
ZCH = 32
EW = CH * NCH
EP = NW * EW
N_PAD = 10240
ZB = N_PAD // NS

_MESH = plsc.VectorSubcoreMesh(core_axis_name="c", subcore_axis_name="s")

BM = 2000
GRID = N // BM
BME = 2000
GRIDE = N // BME


def _fill_rows(ref, nrows, value):
    @pl.loop(0, nrows)
    def _(i):
        ref.at[pl.ds(i, 1), :][...] = jnp.full((1, 16), value, jnp.float32)


@functools.partial(
    pl.kernel,
    out_type=jax.ShapeDtypeStruct((NC, N_PAD, 16), jnp.float32),
    mesh=_MESH,
    scratch_types=[
        pltpu.VMEM((NCH, CH), jnp.int32),
        pltpu.VMEM((CH, 16), jnp.float32),
        pltpu.VMEM((ZB, 16), jnp.float32),
        pltpu.VMEM_SHARED((N_PAD, 16), jnp.float32),
        pltpu.SemaphoreType.DMA,
    ],
    compiler_params=pltpu.CompilerParams(use_tc_tiling_on_sc=False),
)
def _deg_kernel(ei4_hbm, out_hbm, didx, ones_v, zeros_v, acc, ssem):
    c = lax.axis_index("c")
    s = lax.axis_index("s")
    wid = s * NC + c

    _fill_rows(ones_v, CH, 1.0)
    _fill_rows(zeros_v, ZB, 0.0)
    pltpu.sync_copy(zeros_v, acc.at[pl.ds(s * ZB, ZB)])
    plsc.subcore_barrier()

    pltpu.sync_copy(ei4_hbm.at[1, wid], didx)

    for j in range(8):
        pltpu.async_copy(ones_v, acc.at[didx.at[j]], ssem, add=True)

    @pl.loop(8, NCH, step=8)
    def _(g):
        for j in range(8):
            pltpu.async_copy(ones_v, acc.at[didx.at[g + j]], ssem, add=True)
        for j in range(8):
            pltpu.make_async_copy(ones_v, acc.at[didx.at[j]], ssem).wait()

    for j in range(8):
        pltpu.make_async_copy(ones_v, acc.at[didx.at[j]], ssem).wait()

    plsc.subcore_barrier()
    pltpu.sync_copy(acc.at[pl.ds(s * ZB, ZB)],
                    out_hbm.at[c, pl.ds(s * ZB, ZB)])


@functools.partial(
    pl.kernel,
    out_type=jax.ShapeDtypeStruct((NC, N_PAD, DH), jnp.float32),
    mesh=_MESH,
    scratch_types=[
        pltpu.VMEM((NCH, CH), jnp.int32),
        pltpu.VMEM((NCH, CH), jnp.int32),
        pltpu.VMEM((4, CH, DH), jnp.float32),
        pltpu.VMEM((ZCH, DH), jnp.float32),
        pltpu.VMEM_SHARED((N_PAD, DH), jnp.float32),
        pltpu.VMEM_SHARED((N_PAD, DH), jnp.float32),
        pltpu.SemaphoreType.DMA((4,)),
        pltpu.SemaphoreType.DMA((4,)),
    ],
    compiler_params=pltpu.CompilerParams(use_tc_tiling_on_sc=False),
)
def _agg_kernel(hs_hbm, ei4_hbm, out_hbm,
                sidx, didx, rows, zeros_v, acc, hs_sh, gsem, ssem):
    c = lax.axis_index("c")
    s = lax.axis_index("s")
    wid = s * NC + c

    @pl.loop(0, ZCH)
    def _(i):
        @pl.loop(0, DH, step=16)
        def _(j):
            zeros_v.at[pl.ds(i, 1), pl.ds(j, 16)][...] = jnp.zeros(
                (1, 16), jnp.float32)

    pltpu.async_copy(hs_hbm.at[pl.ds(s * (N // NS), N // NS)],
                     hs_sh.at[pl.ds(s * (N // NS), N // NS)], gsem.at[0])
    for z in range(ZB // ZCH):
        pltpu.async_copy(zeros_v, acc.at[pl.ds(s * ZB + z * ZCH, ZCH)],
                         ssem.at[0])
    pltpu.make_async_copy(hs_hbm.at[pl.ds(s * (N // NS), N // NS)],
                          hs_sh.at[pl.ds(s * (N // NS), N // NS)],
                          gsem.at[0]).wait()
    for z in range(ZB // ZCH):
        pltpu.make_async_copy(zeros_v, acc.at[pl.ds(s * ZB + z * ZCH, ZCH)],
                              ssem.at[0]).wait()
    plsc.subcore_barrier()

    pltpu.sync_copy(ei4_hbm.at[0, wid], sidx)
    pltpu.sync_copy(ei4_hbm.at[1, wid], didx)

    def fire_gather(k, b):
        pltpu.async_copy(hs_sh.at[sidx.at[k]], rows.at[b], gsem.at[b])

    def wait_gather(k, b):
        pltpu.make_async_copy(hs_sh.at[sidx.at[k]], rows.at[b],
                              gsem.at[b]).wait()

    def fire_scatter(k, b):
        pltpu.async_copy(rows.at[b], acc.at[didx.at[k]], ssem.at[b], add=True)

    def wait_scatter(k, b):
        pltpu.make_async_copy(rows.at[b], acc.at[didx.at[k]],
                              ssem.at[b]).wait()

    fire_gather(0, 0)
    fire_gather(1, 1)
    fire_gather(2, 2)
    wait_gather(0, 0)
    fire_scatter(0, 0)
    fire_gather(3, 3)
    wait_gather(1, 1)
    fire_scatter(1, 1)
    wait_scatter(0, 0)
    fire_gather(4, 0)
    wait_gather(2, 2)
    fire_scatter(2, 2)
    wait_scatter(1, 1)
    fire_gather(5, 1)
    wait_gather(3, 3)
    fire_scatter(3, 3)

    @pl.loop(4, NCH - 4, step=4)
    def _(kk):
        for j in range(4):
            k = kk + j
            bn = (j + 2) % 4
            wait_scatter(k - 2, bn)
            fire_gather(k + 2, bn)
            wait_gather(k, j)
            fire_scatter(k, j)

    k0 = NCH - 4
    wait_scatter(k0 - 2, 2)
    fire_gather(k0 + 2, 2)
    wait_gather(k0, 0)
    fire_scatter(k0, 0)
    wait_scatter(k0 - 1, 3)
    fire_gather(k0 + 3, 3)
    wait_gather(k0 + 1, 1)
    fire_scatter(k0 + 1, 1)
    wait_gather(k0 + 2, 2)
    fire_scatter(k0 + 2, 2)
    wait_gather(k0 + 3, 3)
    fire_scatter(k0 + 3, 3)
    wait_scatter(k0, 0)
    wait_scatter(k0 + 1, 1)
    wait_scatter(k0 + 2, 2)
    wait_scatter(k0 + 3, 3)

    plsc.subcore_barrier()
    pltpu.sync_copy(acc.at[pl.ds(s * ZB, ZB)],
                    out_hbm.at[c, pl.ds(s * ZB, ZB)])


_PREC = lax.Precision.HIGHEST


def _tc0_body(x_ref, w_ref, h_ref):
    h_ref[...] = jnp.dot(x_ref[...], w_ref[...], precision=_PREC,
                         preferred_element_type=jnp.float32)


def _tc1_body(h_ref, dp_ref, hs_ref, dinv_ref):
    deg = dp_ref[0] + dp_ref[1] + 1.0
    dinv = lax.rsqrt(deg)
    hs_ref[...] = h_ref[...] * dinv[:, 0:1]
    dinv_ref[...] = dinv


def _tc2_body(ap_ref, hs_ref, dinv_ref, b_ref, w_ref, out_ref):
    di = dinv_ref[:, 0:1]
    t = di * (ap_ref[0] + ap_ref[1] + hs_ref[...]) + b_ref[...]
    t = jnp.maximum(t, 0.0)
    out_ref[...] = jnp.dot(t, w_ref[...], precision=_PREC,
                           preferred_element_type=jnp.float32) * di


def _tc3_body(ap_ref, hs_ref, dinv_ref, b_ref, w3_ref, b3_ref, out_ref):
    di = dinv_ref[:, 0:1]
    t = di * (ap_ref[0] + ap_ref[1] + hs_ref[...]) + b_ref[...]
    t = jnp.maximum(t, 0.0)
    logits = jnp.dot(t, w3_ref[...], precision=_PREC,
                     preferred_element_type=jnp.float32) + b3_ref[...]
    l0 = logits[:, 0:1]
    l1 = logits[:, 1:2]
    m = jnp.maximum(l0, l1)
    lse = m + jnp.log(jnp.exp(l0 - m) + jnp.exp(l1 - m))
    out_ref[...] = jnp.concatenate([l0 - lse, l1 - lse], axis=1)


_tc0 = pl.pallas_call(
    _tc0_body,
    grid=(GRID,),
    in_specs=[
        pl.BlockSpec((BM, D_IN), lambda i: (i, 0)),
        pl.BlockSpec((D_IN, DH), lambda i: (0, 0)),
    ],
    out_specs=pl.BlockSpec((BM, DH), lambda i: (i, 0)),
    out_shape=jax.ShapeDtypeStruct((N, DH), jnp.float32),
)

_tc1 = pl.pallas_call(
    _tc1_body,
    grid=(GRIDE,),
    in_specs=[
        pl.BlockSpec((BME, DH), lambda i: (i, 0)),
        pl.BlockSpec((NC, BME, 16), lambda i: (0, i, 0)),
    ],
    out_specs=[
        pl.BlockSpec((BME, DH), lambda i: (i, 0)),
        pl.BlockSpec((BME, 16), lambda i: (i, 0)),
    ],
    out_shape=[
        jax.ShapeDtypeStruct((N, DH), jnp.float32),
        jax.ShapeDtypeStruct((N, 16), jnp.float32),
    ],
)

_tc2 = pl.pallas_call(
    _tc2_body,
    grid=(GRIDE,),
    in_specs=[
        pl.BlockSpec((NC, BME, DH), lambda i: (0, i, 0)),
        pl.BlockSpec((BME, DH), lambda i: (i, 0)),
        pl.BlockSpec((BME, 16), lambda i: (i, 0)),
        pl.BlockSpec((1, DH), lambda i: (0, 0)),
        pl.BlockSpec((DH, DH), lambda i: (0, 0)),
    ],
    out_specs=pl.BlockSpec((BME, DH), lambda i: (i, 0)),
    out_shape=jax.ShapeDtypeStruct((N, DH), jnp.float32),
)

_tc3 = pl.pallas_call(
    _tc3_body,
    grid=(GRIDE,),
    in_specs=[
        pl.BlockSpec((NC, BME, DH), lambda i: (0, i, 0)),
        pl.BlockSpec((BME, DH), lambda i: (i, 0)),
        pl.BlockSpec((BME, 16), lambda i: (i, 0)),
        pl.BlockSpec((1, DH), lambda i: (0, 0)),
        pl.BlockSpec((DH, 128), lambda i: (0, 0)),
        pl.BlockSpec((1, 128), lambda i: (0, 0)),
    ],
    out_specs=pl.BlockSpec((BME, 2), lambda i: (i, 0)),
    out_shape=jax.ShapeDtypeStruct((N, 2), jnp.float32),
)


def kernel(x, edge_index, W1, b1, W2, b2, W3, b3):
    ei4 = jnp.pad(edge_index, ((0, 0), (0, EP - E)),
                  constant_values=N).reshape(2, NW, NCH, CH)

    dp = _deg_kernel(ei4)
    h1 = _tc0(x, W1)
    hs1, dinv = _tc1(h1, dp)
    ap1 = _agg_kernel(hs1, ei4)
    hs2 = _tc2(ap1, hs1, dinv, b1.reshape(1, DH), W2)
    ap2 = _agg_kernel(hs2, ei4)
    w3p = jnp.pad(W3, ((0, 0), (0, 128 - W3.shape[1])))
    b3p = jnp.pad(b3, (0, 128 - b3.shape[0])).reshape(1, 128)
    return _tc3(ap2, hs2, dinv, b2.reshape(1, DH), w3p, b3p)


# speedup vs baseline: 1.0416x; 1.0030x over previous
import functools

import jax
import jax.numpy as jnp
from jax import lax
from jax.experimental import pallas as pl
from jax.experimental.pallas import tpu as pltpu
from jax.experimental.pallas import tpu_sc as plsc

N = 10000
E = 320000
D_IN = 128
DH = 64

NC = 2
NS = 16
NW = NC * NS
CH = 128
NCH = 80
GS = 8
NG = NCH // GS
ZCH = 32
EW = CH * NCH
EP = NW * EW
N_PAD = 10240
ZB = N_PAD // NS

_MESH = plsc.VectorSubcoreMesh(core_axis_name="c", subcore_axis_name="s")

BM = 2000
GRID = N // BM
BME = 2000
GRIDE = N // BME


def _fill_rows(ref, nrows, value):
    @pl.loop(0, nrows)
    def _(i):
        ref.at[pl.ds(i, 1), :][...] = jnp.full((1, 16), value, jnp.float32)


@functools.partial(
    pl.kernel,
    out_type=jax.ShapeDtypeStruct((NC, N_PAD, 16), jnp.float32),
    mesh=_MESH,
    scratch_types=[
        pltpu.VMEM((NCH, CH), jnp.int32),
        pltpu.VMEM((CH, 16), jnp.float32),
        pltpu.VMEM((ZB, 16), jnp.float32),
        pltpu.VMEM_SHARED((N_PAD, 16), jnp.float32),
        pltpu.SemaphoreType.DMA,
    ],
    compiler_params=pltpu.CompilerParams(use_tc_tiling_on_sc=False),
)
def _deg_kernel(ei4_hbm, out_hbm, didx, ones_v, zeros_v, acc, ssem):
    c = lax.axis_index("c")
    s = lax.axis_index("s")
    wid = s * NC + c

    _fill_rows(ones_v, CH, 1.0)
    _fill_rows(zeros_v, ZB, 0.0)
    pltpu.sync_copy(zeros_v, acc.at[pl.ds(s * ZB, ZB)])
    plsc.subcore_barrier()

    pltpu.sync_copy(ei4_hbm.at[1, wid], didx)

    for j in range(8):
        pltpu.async_copy(ones_v, acc.at[didx.at[j]], ssem, add=True)

    @pl.loop(8, NCH, step=8)
    def _(g):
        for j in range(8):
            pltpu.async_copy(ones_v, acc.at[didx.at[g + j]], ssem, add=True)
        for j in range(8):
            pltpu.make_async_copy(ones_v, acc.at[didx.at[j]], ssem).wait()

    for j in range(8):
        pltpu.make_async_copy(ones_v, acc.at[didx.at[j]], ssem).wait()

    plsc.subcore_barrier()
    pltpu.sync_copy(acc.at[pl.ds(s * ZB, ZB)],
                    out_hbm.at[c, pl.ds(s * ZB, ZB)])


@functools.partial(
    pl.kernel,
    out_type=jax.ShapeDtypeStruct((NC, N_PAD, DH), jnp.float32),
    mesh=_MESH,
    scratch_types=[
        pltpu.VMEM((2, GS, CH), jnp.int32),
        pltpu.VMEM((2, GS, CH), jnp.int32),
        pltpu.VMEM((4, CH, DH), jnp.float32),
        pltpu.VMEM((ZCH, DH), jnp.float32),
        pltpu.VMEM_SHARED((N_PAD, DH), jnp.float32),
        pltpu.VMEM_SHARED((N_PAD, DH), jnp.float32),
        pltpu.SemaphoreType.DMA((4,)),
        pltpu.SemaphoreType.DMA((4,)),
        pltpu.SemaphoreType.DMA,
    ],
    compiler_params=pltpu.CompilerParams(use_tc_tiling_on_sc=False),
)
def _agg_kernel(hs_hbm, ei4_hbm, out_hbm,
                sidx, didx, rows, zeros_v, acc, hs_sh, gsem, ssem, isem):
    c = lax.axis_index("c")
    s = lax.axis_index("s")
    wid = s * NC + c

    @pl.loop(0, ZCH)
    def _(i):
        @pl.loop(0, DH, step=16)
        def _(j):
            zeros_v.at[pl.ds(i, 1), pl.ds(j, 16)][...] = jnp.zeros(
                (1, 16), jnp.float32)

    pltpu.async_copy(hs_hbm.at[pl.ds(s * (N // NS), N // NS)],
                     hs_sh.at[pl.ds(s * (N // NS), N // NS)], gsem.at[0])
    for z in range(ZB // ZCH):
        pltpu.async_copy(zeros_v, acc.at[pl.ds(s * ZB + z * ZCH, ZCH)],
                         ssem.at[0])
    pltpu.make_async_copy(hs_hbm.at[pl.ds(s * (N // NS), N // NS)],
                          hs_sh.at[pl.ds(s * (N // NS), N // NS)],
                          gsem.at[0]).wait()
    for z in range(ZB // ZCH):
        pltpu.make_async_copy(zeros_v, acc.at[pl.ds(s * ZB + z * ZCH, ZCH)],
                              ssem.at[0]).wait()
    plsc.subcore_barrier()

    def idx_fire(g, p):
        pltpu.async_copy(ei4_hbm.at[0, wid, pl.ds(g * GS, GS)],
                         sidx.at[p], isem)
        pltpu.async_copy(ei4_hbm.at[1, wid, pl.ds(g * GS, GS)],
                         didx.at[p], isem)

    def idx_wait():
        pltpu.make_async_copy(ei4_hbm.at[0, wid, pl.ds(0, GS)],
                              sidx.at[0], isem).wait()
        pltpu.make_async_copy(ei4_hbm.at[1, wid, pl.ds(0, GS)],
                              didx.at[0], isem).wait()

    def fire_gather(p, j, b):
        pltpu.async_copy(hs_sh.at[sidx.at[p, j]], rows.at[b], gsem.at[b])

    def wait_gather(b):
        pltpu.make_async_copy(hs_sh.at[sidx.at[0, 0]], rows.at[b],
                              gsem.at[b]).wait()

    def fire_scatter(p, j, b):
        pltpu.async_copy(rows.at[b], acc.at[didx.at[p, j]], ssem.at[b],
                         add=True)

    def wait_scatter(b):
        pltpu.make_async_copy(rows.at[b], acc.at[didx.at[0, 0]],
                              ssem.at[b]).wait()

    pltpu.sync_copy(ei4_hbm.at[0, wid, pl.ds(0, GS)], sidx.at[0])
    pltpu.sync_copy(ei4_hbm.at[1, wid, pl.ds(0, GS)], didx.at[0])
    idx_fire(1, 1)

    fire_gather(0, 0, 0)
    fire_gather(0, 1, 1)
    for j in range(GS):
        b = j % 4
        bn = (j + 2) % 4
        if j >= 2:
            wait_scatter(bn)
        if j == 6:
            idx_wait()
        if j < 6:
            fire_gather(0, j + 2, bn)
        else:
            fire_gather(1, j - 6, bn)
        wait_gather(b)
        fire_scatter(0, j, b)

    @pl.loop(GS, (NG - 1) * GS, step=GS)
    def _(kk):
        g = kk // GS
        p = lax.rem(g, 2)
        p1 = lax.rem(g + 1, 2)
        for j in range(GS):
            b = j % 4
            bn = (j + 2) % 4
            wait_scatter(bn)
            if j == 2:
                idx_fire(g + 1, p1)
            if j == 6:
                idx_wait()
            if j < 6:
                fire_gather(p, j + 2, bn)
            else:
                fire_gather(p1, j - 6, bn)
            wait_gather(b)
            fire_scatter(p, j, b)

    pe = (NG - 1) % 2
    for j in range(GS):
        b = j % 4
        bn = (j + 2) % 4
        if j < 6:
            wait_scatter(bn)
            fire_gather(pe, j + 2, bn)
        wait_gather(b)
        fire_scatter(pe, j, b)
    for b in range(4):
        wait_scatter(b)

    plsc.subcore_barrier()
    pltpu.sync_copy(acc.at[pl.ds(s * ZB, ZB)],
                    out_hbm.at[c, pl.ds(s * ZB, ZB)])


_PREC = lax.Precision.HIGHEST


def _tc0_body(x_ref, w_ref, h_ref):
    h_ref[...] = jnp.dot(x_ref[...], w_ref[...], precision=_PREC,
                         preferred_element_type=jnp.float32)


def _tc1_body(h_ref, dp_ref, hs_ref, dinv_ref):
    deg = dp_ref[0] + dp_ref[1] + 1.0
    dinv = lax.rsqrt(deg)
    hs_ref[...] = h_ref[...] * dinv[:, 0:1]
    dinv_ref[...] = dinv


def _tc2_body(ap_ref, hs_ref, dinv_ref, b_ref, w_ref, out_ref):
    di = dinv_ref[:, 0:1]
    t = di * (ap_ref[0] + ap_ref[1] + hs_ref[...]) + b_ref[...]
    t = jnp.maximum(t, 0.0)
    out_ref[...] = jnp.dot(t, w_ref[...], precision=_PREC,
                           preferred_element_type=jnp.float32) * di


def _tc3_body(ap_ref, hs_ref, dinv_ref, b_ref, w3_ref, b3_ref, out_ref):
    di = dinv_ref[:, 0:1]
    t = di * (ap_ref[0] + ap_ref[1] + hs_ref[...]) + b_ref[...]
    t = jnp.maximum(t, 0.0)
    logits = jnp.dot(t, w3_ref[...], precision=_PREC,
                     preferred_element_type=jnp.float32) + b3_ref[...]
    l0 = logits[:, 0:1]
    l1 = logits[:, 1:2]
    m = jnp.maximum(l0, l1)
    lse = m + jnp.log(jnp.exp(l0 - m) + jnp.exp(l1 - m))
    out_ref[...] = jnp.concatenate([l0 - lse, l1 - lse], axis=1)


_tc0 = pl.pallas_call(
    _tc0_body,
    grid=(GRID,),
    in_specs=[
        pl.BlockSpec((BM, D_IN), lambda i: (i, 0)),
        pl.BlockSpec((D_IN, DH), lambda i: (0, 0)),
    ],
    out_specs=pl.BlockSpec((BM, DH), lambda i: (i, 0)),
    out_shape=jax.ShapeDtypeStruct((N, DH), jnp.float32),
)

_tc1 = pl.pallas_call(
    _tc1_body,
    grid=(GRIDE,),
    in_specs=[
        pl.BlockSpec((BME, DH), lambda i: (i, 0)),
        pl.BlockSpec((NC, BME, 16), lambda i: (0, i, 0)),
    ],
    out_specs=[
        pl.BlockSpec((BME, DH), lambda i: (i, 0)),
        pl.BlockSpec((BME, 16), lambda i: (i, 0)),
    ],
    out_shape=[
        jax.ShapeDtypeStruct((N, DH), jnp.float32),
        jax.ShapeDtypeStruct((N, 16), jnp.float32),
    ],
)

_tc2 = pl.pallas_call(
    _tc2_body,
    grid=(GRIDE,),
    in_specs=[
        pl.BlockSpec((NC, BME, DH), lambda i: (0, i, 0)),
        pl.BlockSpec((BME, DH), lambda i: (i, 0)),
        pl.BlockSpec((BME, 16), lambda i: (i, 0)),
        pl.BlockSpec((1, DH), lambda i: (0, 0)),
        pl.BlockSpec((DH, DH), lambda i: (0, 0)),
    ],
    out_specs=pl.BlockSpec((BME, DH), lambda i: (i, 0)),
    out_shape=jax.ShapeDtypeStruct((N, DH), jnp.float32),
)

_tc3 = pl.pallas_call(
    _tc3_body,
    grid=(GRIDE,),
    in_specs=[
        pl.BlockSpec((NC, BME, DH), lambda i: (0, i, 0)),
        pl.BlockSpec((BME, DH), lambda i: (i, 0)),
        pl.BlockSpec((BME, 16), lambda i: (i, 0)),
        pl.BlockSpec((1, DH), lambda i: (0, 0)),
        pl.BlockSpec((DH, 128), lambda i: (0, 0)),
        pl.BlockSpec((1, 128), lambda i: (0, 0)),
    ],
    out_specs=pl.BlockSpec((BME, 2), lambda i: (i, 0)),
    out_shape=jax.ShapeDtypeStruct((N, 2), jnp.float32),
)


def kernel(x, edge_index, W1, b1, W2, b2, W3, b3):
    ei4 = jnp.pad(edge_index, ((0, 0), (0, EP - E)),
                  constant_values=N).reshape(2, NW, NCH, CH)

    dp = _deg_kernel(ei4)
    h1 = _tc0(x, W1)
    hs1, dinv = _tc1(h1, dp)
    ap1 = _agg_kernel(hs1, ei4)
    hs2 = _tc2(ap1, hs1, dinv, b1.reshape(1, DH), W2)
    ap2 = _agg_kernel(hs2, ei4)
    w3p = jnp.pad(W3, ((0, 0), (0, 128 - W3.shape[1])))
    b3p = jnp.pad(b3, (0, 128 - b3.shape[0])).reshape(1, 128)
    return _tc3(ap2, hs2, dinv, b2.reshape(1, DH), w3p, b3p)
